# baseline (speedup 1.0000x reference)
import functools
import math

import jax
import jax.numpy as jnp
from jax import lax
from jax.experimental import pallas as pl
from jax.experimental.pallas import tpu as pltpu
from jax.experimental.pallas import tpu_sc as plsc

D_MODEL = 64
SCALE = math.sqrt(D_MODEL)

NUM_CORES = 2
NUM_SUBCORES = 16
NUM_WORKERS = NUM_CORES * NUM_SUBCORES
NBUF = 4
ROWS_PER_STEP = 5


@functools.partial(jax.jit, static_argnums=(2, 3))
def _emb_lookup(x, table, batch, seq):
  rows_per_w = batch // NUM_WORKERS
  assert rows_per_w % NBUF == 0 and rows_per_w // NBUF >= 2
  n_rounds = rows_per_w // NBUF
  mesh = plsc.VectorSubcoreMesh(core_axis_name="c", subcore_axis_name="s")

  scratch = [pltpu.VMEM((rows_per_w, seq), jnp.int32)]
  scratch += [pltpu.VMEM((seq, D_MODEL), jnp.float32) for _ in range(2 * NBUF)]
  scratch += [pltpu.SemaphoreType.DMA for _ in range(2 * NBUF)]

  @functools.partial(
      pl.kernel,
      mesh=mesh,
      out_type=jax.ShapeDtypeStruct((batch, seq, D_MODEL), jnp.float32),
      scratch_types=scratch,
      compiler_params=pltpu.CompilerParams(use_tc_tiling_on_sc=False),
  )
  def k(x_hbm, table_hbm, out_hbm, idx_v, *bufs_and_sems):
    in_bufs = bufs_and_sems[:NBUF]
    out_bufs = bufs_and_sems[NBUF:2 * NBUF]
    g_sems = bufs_and_sems[2 * NBUF:3 * NBUF]
    s_sems = bufs_and_sems[3 * NBUF:4 * NBUF]
    wid = lax.axis_index("s") * NUM_CORES + lax.axis_index("c")
    base = wid * rows_per_w

    pltpu.sync_copy(x_hbm.at[pl.ds(base, rows_per_w)], idx_v)

    def fire_gather(c, b):
      pltpu.async_copy(table_hbm.at[idx_v.at[c]], in_bufs[b], g_sems[b])

    def wait_gather(c, b):
      pltpu.make_async_copy(
          table_hbm.at[idx_v.at[c]], in_bufs[b], g_sems[b]).wait()

    def fire_scatter(c, b):
      pltpu.async_copy(out_bufs[b], out_hbm.at[base + c], s_sems[b])

    def wait_scatter(c, b):
      pltpu.make_async_copy(
          out_bufs[b], out_hbm.at[base + c], s_sems[b]).wait()

    def scale(b):
      src, dst = in_bufs[b], out_bufs[b]

      def body(r, carry):
        for rr in range(ROWS_PER_STEP):
          for kk in range(D_MODEL // 16):
            sl = (r * ROWS_PER_STEP + rr, pl.ds(kk * 16, 16))
            dst[sl] = src[sl] * SCALE
        return carry

      lax.fori_loop(0, seq // ROWS_PER_STEP, body, 0, unroll=False)

    for b in range(NBUF):
      fire_gather(b, b)

    for b in range(NBUF):
      wait_gather(b, b)
      scale(b)
      fire_gather(NBUF + b, b)
      fire_scatter(b, b)

    def outer(i, carry):
      c0 = i * NBUF
      for b in range(NBUF):
        wait_gather(c0 + b, b)
        wait_scatter(c0 - NBUF + b, b)
        scale(b)
        fire_gather(c0 + NBUF + b, b)
        fire_scatter(c0 + b, b)
      return carry

    lax.fori_loop(1, n_rounds - 1, outer, 0, unroll=False)

    c0 = (n_rounds - 1) * NBUF
    for b in range(NBUF):
      wait_gather(c0 + b, b)
      wait_scatter(c0 - NBUF + b, b)
      scale(b)
      fire_scatter(c0 + b, b)

    for b in range(NBUF):
      wait_scatter(c0 + b, b)

  return k(x, table)


def kernel(x, emb_table):
  batch, seq = x.shape
  assert batch % NUM_WORKERS == 0 and seq % ROWS_PER_STEP == 0
  return _emb_lookup(x.astype(jnp.int32), emb_table, batch, seq)


# speedup vs baseline: 1.0412x; 1.0412x over previous
import functools
import math

import jax
import jax.numpy as jnp
from jax import lax
from jax.experimental import pallas as pl
from jax.experimental.pallas import tpu as pltpu
from jax.experimental.pallas import tpu_sc as plsc

D_MODEL = 64
SCALE = math.sqrt(D_MODEL)

NUM_CORES = 2
NUM_SUBCORES = 16
NUM_WORKERS = NUM_CORES * NUM_SUBCORES
NBUF = 4
ROWS_PER_STEP = 5


@functools.partial(jax.jit, static_argnums=(2, 3))
def _emb_lookup(x, table2, batch, seq):
  rows_per_w = batch // NUM_WORKERS
  assert rows_per_w % NBUF == 0 and rows_per_w // NBUF >= 2
  n_rounds = rows_per_w // NBUF
  mesh = plsc.VectorSubcoreMesh(core_axis_name="c", subcore_axis_name="s")

  scratch = [pltpu.VMEM((rows_per_w, seq), jnp.int32)]
  scratch += [pltpu.VMEM((seq, 128), jnp.float32) for _ in range(NBUF)]
  scratch += [pltpu.VMEM((seq, D_MODEL), jnp.float32) for _ in range(NBUF)]
  scratch += [pltpu.SemaphoreType.DMA for _ in range(2 * NBUF)]

  @functools.partial(
      pl.kernel,
      mesh=mesh,
      out_type=jax.ShapeDtypeStruct((batch, seq, D_MODEL), jnp.float32),
      scratch_types=scratch,
  )
  def k(x_hbm, table_hbm, out_hbm, idx_v, *bufs_and_sems):
    in_bufs = bufs_and_sems[:NBUF]
    out_bufs = bufs_and_sems[NBUF:2 * NBUF]
    g_sems = bufs_and_sems[2 * NBUF:3 * NBUF]
    s_sems = bufs_and_sems[3 * NBUF:4 * NBUF]
    wid = lax.axis_index("s") * NUM_CORES + lax.axis_index("c")
    base = wid * rows_per_w

    pltpu.sync_copy(x_hbm.at[pl.ds(base, rows_per_w)], idx_v)

    def fire_gather(c, b):
      pltpu.async_copy(table_hbm.at[idx_v.at[c]], in_bufs[b], g_sems[b])

    def wait_gather(c, b):
      pltpu.make_async_copy(
          table_hbm.at[idx_v.at[c]], in_bufs[b], g_sems[b]).wait()

    def fire_scatter(c, b):
      pltpu.async_copy(out_bufs[b], out_hbm.at[base + c], s_sems[b])

    def wait_scatter(c, b):
      pltpu.make_async_copy(
          out_bufs[b], out_hbm.at[base + c], s_sems[b]).wait()

    def scale(b):
      src, dst = in_bufs[b], out_bufs[b]

      def body(r, carry):
        for rr in range(ROWS_PER_STEP):
          row = r * ROWS_PER_STEP + rr
          for kk in range(D_MODEL // 16):
            dst[row, pl.ds(kk * 16, 16)] = (
                src[row, pl.ds(kk * 16, 16)] * SCALE)
        return carry

      lax.fori_loop(0, seq // ROWS_PER_STEP, body, 0, unroll=False)

    for b in range(NBUF):
      fire_gather(b, b)

    for b in range(NBUF):
      wait_gather(b, b)
      scale(b)
      fire_gather(NBUF + b, b)
      fire_scatter(b, b)

    def outer(i, carry):
      c0 = i * NBUF
      for b in range(NBUF):
        wait_gather(c0 + b, b)
        wait_scatter(c0 - NBUF + b, b)
        scale(b)
        fire_gather(c0 + NBUF + b, b)
        fire_scatter(c0 + b, b)
      return carry

    lax.fori_loop(1, n_rounds - 1, outer, 0, unroll=False)

    c0 = (n_rounds - 1) * NBUF
    for b in range(NBUF):
      wait_gather(c0 + b, b)
      wait_scatter(c0 - NBUF + b, b)
      scale(b)
      fire_scatter(c0 + b, b)

    for b in range(NBUF):
      wait_scatter(c0 + b, b)

  return k(x, table2)


def kernel(x, emb_table):
  batch, seq = x.shape
  assert batch % NUM_WORKERS == 0 and seq % ROWS_PER_STEP == 0
  table2 = jnp.concatenate([emb_table, emb_table], axis=1)
  return _emb_lookup(x.astype(jnp.int32), table2, batch, seq)
